# baseline (speedup 1.0000x reference)
import dataclasses
import functools
import math

import jax
import jax.numpy as jnp
from jax import lax
from jax.experimental import pallas as pl
from jax.experimental.pallas import tpu as pltpu
from jax.experimental.pallas import tpu_sc as plsc

N = 10000
E = 320000
D_IN = 128
H1 = 256
H2 = 256
OUT = 64

def _sc_compiler_params():
    cp = pltpu.CompilerParams()
    if "needs_layout_passes" in pltpu.CompilerParams.__dataclass_fields__:
        cp = dataclasses.replace(cp, needs_layout_passes=False)
    return cp


IB = 128
NCHUNK = E // IB
NCHUNK_PAD = 2512
NTILES = 16
GDEPTH = 2
ROWS_MAIN = 632
ROWS_LAST = N - 15 * ROWS_MAIN
KMAX = (NCHUNK + NTILES - 1) // NTILES


def _pipelined_segsum(table, sd_hbm, idxb, rows_v, isem, gsems, ssems,
                      acc, base, nk, nkmax, deg_fn=None):
    nrows = len(gsems)
    nidx = 4
    unroll = nrows * nidx // math.gcd(nrows, nidx)

    def issue_gather(kk, pr, pi):
        @pl.when(kk < nk)
        def _():
            pltpu.async_copy(table.at[idxb.at[pi].at[0]], rows_v.at[pr],
                             gsems[pr])

    pltpu.sync_copy(sd_hbm.at[base], idxb.at[0])

    @pl.when(1 < nk)
    def _():
        pltpu.sync_copy(sd_hbm.at[base + 1], idxb.at[1])

    issue_gather(0, 0, 0)

    nphase = unroll * ((nkmax + nrows + unroll - 1) // unroll)

    @pl.loop(0, nphase // unroll)
    def _(q):
        kk0 = q * unroll
        for j in range(unroll):
            kk = kk0 + j
            pr = j % nrows
            nx = (j + 1) % nrows
            pi = j % nidx

            @pl.when((kk >= 1) & (kk + 1 < nk))
            def _():
                pltpu.make_async_copy(sd_hbm.at[base + kk + 1],
                                      idxb.at[(pi + 1) % nidx],
                                      isem).wait()

            back = nrows - 1

            @pl.when((kk >= back) & (kk - back < nk))
            def _():
                pltpu.make_async_copy(
                    rows_v.at[nx],
                    acc.at[idxb.at[(pi - back) % nidx].at[1]],
                    ssems[nx]).wait()

            issue_gather(kk + 1, nx, (pi + 1) % nidx)

            @pl.when(kk < nk)
            def _():
                pltpu.make_async_copy(table.at[idxb.at[pi].at[0]],
                                      rows_v.at[pr], gsems[pr]).wait()
                pltpu.async_copy(rows_v.at[pr],
                                 acc.at[idxb.at[pi].at[1]], ssems[pr],
                                 add=True)
                if deg_fn is not None:
                    deg_fn(pi)

            @pl.when(kk + 2 < nk)
            def _():
                pltpu.async_copy(sd_hbm.at[base + kk + 2],
                                 idxb.at[(pi + 2) % nidx], isem)


def _sc_layer1():
    W = D_IN
    mesh = plsc.VectorSubcoreMesh(core_axis_name="c", subcore_axis_name="s")
    out_types = (jax.ShapeDtypeStruct((2, N, W), jnp.float32),
                 jax.ShapeDtypeStruct((2 * NTILES * N,), jnp.float32))
    half = NCHUNK // 2
    nkmax = half // NTILES + 1
    scratch = [
        pltpu.VMEM((4, 2, IB), jnp.int32),
        pltpu.VMEM((2, IB, W), jnp.float32),
        pltpu.VMEM((N,), jnp.float32),
        pltpu.VMEM_SHARED((N, W), jnp.float32),
        pltpu.SemaphoreType.DMA,
        pltpu.SemaphoreType.DMA,
        pltpu.SemaphoreType.DMA,
        pltpu.SemaphoreType.DMA,
        pltpu.SemaphoreType.DMA,
    ]

    def body(table, sd_hbm, zW_hbm, out, deg_out, idxb, rows_v, degp,
             acc, isem, gsem0, gsem1, ssem0, ssem1):
        c = lax.axis_index("c")
        s = lax.axis_index("s")
        r0 = s * ROWS_MAIN

        def rowcopy(src_ref, dst_ref):
            @pl.when(s < NTILES - 1)
            def _():
                pltpu.sync_copy(src_ref.at[pl.ds(r0, ROWS_MAIN)],
                                dst_ref.at[pl.ds(r0, ROWS_MAIN)])

            @pl.when(s == NTILES - 1)
            def _():
                pltpu.sync_copy(src_ref.at[pl.ds(r0, ROWS_LAST)],
                                dst_ref.at[pl.ds(r0, ROWS_LAST)])

        @pl.loop(0, N // 16)
        def _(i):
            degp[pl.ds(i * 16, 16)] = jnp.zeros((16,), jnp.float32)

        rowcopy(zW_hbm, acc)
        plsc.subcore_barrier()

        base = c * half + s * (half // NTILES) + jnp.minimum(s, 2)
        nk = jnp.where(s < 2, nkmax, nkmax - 1)

        def deg_fn(p):
            @pl.loop(0, IB // 16)
            def _(j):
                idx = idxb[p, 1, pl.ds(j * 16, 16)]
                plsc.addupdate_scatter(degp, [idx],
                                       jnp.ones((16,), jnp.float32))

        _pipelined_segsum(table, sd_hbm, idxb, rows_v, isem,
                          (gsem0, gsem1), (ssem0, ssem1), acc, base, nk,
                          nkmax, deg_fn)

        plsc.subcore_barrier()
        rowcopy(acc, out.at[c])
        wid = c * NTILES + s
        pltpu.sync_copy(degp, deg_out.at[pl.ds(wid * N, N)])

    return functools.partial(pl.kernel, mesh=mesh, out_type=out_types,
                             scratch_types=scratch,
                             compiler_params=_sc_compiler_params())(body)


def _sc_layer2():
    W = H1 // 2
    mesh = plsc.VectorSubcoreMesh(core_axis_name="c", subcore_axis_name="s")
    out_types = (jax.ShapeDtypeStruct((N, W), jnp.float32),
                 jax.ShapeDtypeStruct((N, W), jnp.float32))
    nkmax = NCHUNK // NTILES + 1
    scratch = [
        pltpu.VMEM((4, 2, IB), jnp.int32),
        pltpu.VMEM((3, IB, W), jnp.float32),
        pltpu.VMEM_SHARED((N, W), jnp.float32),
    ] + [pltpu.SemaphoreType.DMA] * 7

    def body(tA, tB, sd_hbm, zW_hbm, outA, outB, idxb, rows_v, acc,
             isem, gsem0, gsem1, gsem2, ssem0, ssem1, ssem2):
        c = lax.axis_index("c")
        s = lax.axis_index("s")
        r0 = s * ROWS_MAIN

        def rowcopy(src_ref, dst_ref):
            @pl.when(s < NTILES - 1)
            def _():
                pltpu.sync_copy(src_ref.at[pl.ds(r0, ROWS_MAIN)],
                                dst_ref.at[pl.ds(r0, ROWS_MAIN)])

            @pl.when(s == NTILES - 1)
            def _():
                pltpu.sync_copy(src_ref.at[pl.ds(r0, ROWS_LAST)],
                                dst_ref.at[pl.ds(r0, ROWS_LAST)])

        base = s * (NCHUNK // NTILES) + jnp.minimum(s, 4)
        nk = jnp.where(s < 4, nkmax, nkmax - 1)

        def core_prog(table, out):
            rowcopy(zW_hbm, acc)
            plsc.subcore_barrier()

            _pipelined_segsum(table, sd_hbm, idxb, rows_v, isem,
                              (gsem0, gsem1, gsem2),
                              (ssem0, ssem1, ssem2), acc,
                              base, nk, nkmax)

            plsc.subcore_barrier()
            rowcopy(acc, out)

        @pl.when(c == 0)
        def _():
            core_prog(tA, outA)

        @pl.when(c == 1)
        def _():
            core_prog(tB, outB)

    return functools.partial(pl.kernel, mesh=mesh, out_type=out_types,
                             scratch_types=scratch,
                             compiler_params=_sc_compiler_params())(body)


def _tc_layer1(x, part, dpart, W1l, W1r, b1):
    RB = 2000

    def body(x_r, p_r, d_r, wl_r, wr_r, b_r, hA_r, hB_r):
        deg = jnp.sum(d_r[...], axis=1, keepdims=True)
        inv = 1.0 / jnp.maximum(deg, 1.0)
        mean = (p_r[0] + p_r[1]) * inv
        h = (jnp.dot(mean, wl_r[...], preferred_element_type=jnp.float32)
             + jnp.dot(x_r[...], wr_r[...], preferred_element_type=jnp.float32)
             + b_r[...])
        h = jnp.maximum(h, 0.0)
        hA_r[...] = h[:, :H1 // 2]
        hB_r[...] = h[:, H1 // 2:]

    grid = (N // RB,)
    row_spec = lambda w: pl.BlockSpec((RB, w), lambda i: (i, 0))
    stk_spec = lambda w: pl.BlockSpec((2, RB, w), lambda i: (0, i, 0))
    full_spec = lambda a, b: pl.BlockSpec((a, b), lambda i: (0, 0))
    return pl.pallas_call(
        body,
        grid=grid,
        in_specs=[row_spec(D_IN), stk_spec(D_IN), row_spec(2 * NTILES),
                  full_spec(D_IN, H1), full_spec(D_IN, H1), full_spec(1, H1)],
        out_specs=[row_spec(H1 // 2), row_spec(H1 // 2)],
        out_shape=[jax.ShapeDtypeStruct((N, H1 // 2), jnp.float32),
                   jax.ShapeDtypeStruct((N, H1 // 2), jnp.float32)],
    )(x, part, dpart, W1l, W1r, b1)


def _tc_layer2_final(hA, hB, aA, aB, dpart, W2l, W2r, b2, Wlin, blin):
    RB = 2000

    def body(hA_r, hB_r, aA_r, aB_r, d_r, wl_r, wr_r, b_r,
             wlin_r, blin_r, out_r):
        deg = jnp.sum(d_r[...], axis=1, keepdims=True)
        inv = 1.0 / jnp.maximum(deg, 1.0)
        mean = jnp.concatenate([aA_r[...], aB_r[...]], axis=1) * inv
        h1 = jnp.concatenate([hA_r[...], hB_r[...]], axis=1)
        h2 = (jnp.dot(mean, wl_r[...], preferred_element_type=jnp.float32)
              + jnp.dot(h1, wr_r[...], preferred_element_type=jnp.float32)
              + b_r[...])
        h2 = jnp.maximum(h2, 0.0)
        o = jnp.dot(h2, wlin_r[...], preferred_element_type=jnp.float32) \
            + blin_r[...]
        m = jnp.max(o, axis=1, keepdims=True)
        lse = m + jnp.log(jnp.sum(jnp.exp(o - m), axis=1, keepdims=True))
        out_r[...] = o - lse

    grid = (N // RB,)
    row_spec = lambda w: pl.BlockSpec((RB, w), lambda i: (i, 0))
    stk_spec = lambda w: pl.BlockSpec((2, RB, w), lambda i: (0, i, 0))
    full_spec = lambda a, b: pl.BlockSpec((a, b), lambda i: (0, 0))
    return pl.pallas_call(
        body,
        grid=grid,
        in_specs=[row_spec(H1 // 2), row_spec(H1 // 2),
                  row_spec(H2 // 2), row_spec(H2 // 2), row_spec(2 * NTILES),
                  full_spec(H1, H2), full_spec(H1, H2), full_spec(1, H2),
                  full_spec(H2, OUT), full_spec(1, OUT)],
        out_specs=row_spec(OUT),
        out_shape=jax.ShapeDtypeStruct((N, OUT), jnp.float32),
    )(hA, hB, aA, aB, dpart, W2l, W2r, b2, Wlin, blin)


def kernel(x, edge_index, W1l, W1r, b1, W2l, W2r, b2, Wlin, blin):
    src = edge_index[0].astype(jnp.int32)
    dst = edge_index[1].astype(jnp.int32)
    sd = jnp.stack([src.reshape(NCHUNK, IB), dst.reshape(NCHUNK, IB)], axis=1)
    sd = jnp.concatenate(
        [sd, jnp.zeros((NCHUNK_PAD - NCHUNK, 2, IB), jnp.int32)], axis=0)

    z128 = jnp.zeros((N, 128), jnp.float32)
    z16 = jnp.zeros((N, 16), jnp.float32)

    part, dego = _sc_layer1()(x, sd, z128)
    dpart = dego.reshape(2 * NTILES, N).T

    hA, hB = _tc_layer1(x, part, dpart, W1l, W1r, b1.reshape(1, H1))

    agg2A, agg2B = _sc_layer2()(hA, hB, sd, z128)

    return _tc_layer2_final(hA, hB, agg2A, agg2B, dpart, W2l, W2r,
                            b2.reshape(1, H2), Wlin, blin.reshape(1, OUT))


# speedup vs baseline: 1.0059x; 1.0059x over previous
import dataclasses
import functools

import jax
import jax.numpy as jnp
from jax import lax
from jax.experimental import pallas as pl
from jax.experimental.pallas import tpu as pltpu
from jax.experimental.pallas import tpu_sc as plsc

N = 10000
E = 320000
D_IN = 128
H1 = 256
H2 = 256
OUT = 64

def _sc_compiler_params():
    cp = pltpu.CompilerParams()
    if "needs_layout_passes" in pltpu.CompilerParams.__dataclass_fields__:
        cp = dataclasses.replace(cp, needs_layout_passes=False)
    return cp


IB = 128
NCHUNK = E // IB
NCHUNK_PAD = 2512
NTILES = 16
ROWS_MAIN = 632
ROWS_LAST = N - 15 * ROWS_MAIN
KMAX = (NCHUNK + NTILES - 1) // NTILES


def _pipelined_segsum(table, sd_hbm, idxb, rows_v, isem, gsems, acc,
                      base, nk, nkmax, deg_fn=None):
    ssems = gsems[2:]
    gsems = gsems[:2]

    def issue_gather(kk, pr, pi):
        @pl.when(kk < nk)
        def _():
            pltpu.async_copy(table.at[idxb.at[pi].at[0]], rows_v.at[pr],
                             gsems[pr])

    pltpu.sync_copy(sd_hbm.at[base], idxb.at[0])

    @pl.when(1 < nk)
    def _():
        pltpu.sync_copy(sd_hbm.at[base + 1], idxb.at[1])

    issue_gather(0, 0, 0)

    nphase = 4 * ((nkmax + 1 + 3) // 4)

    @pl.loop(0, nphase // 4)
    def _(q):
        kk0 = q * 4
        for j in range(4):
            kk = kk0 + j
            pr = j % 2
            o = 1 - pr
            pi = j % 4

            @pl.when((kk >= 1) & (kk + 1 < nk))
            def _():
                pltpu.make_async_copy(sd_hbm.at[base + kk + 1],
                                      idxb.at[(pi + 1) % 4], isem).wait()

            @pl.when((kk >= 1) & (kk - 1 < nk))
            def _():
                pltpu.make_async_copy(
                    rows_v.at[o], acc.at[idxb.at[(pi + 3) % 4].at[1]],
                    ssems[o]).wait()

            issue_gather(kk + 1, o, (pi + 1) % 4)

            @pl.when(kk < nk)
            def _():
                pltpu.make_async_copy(table.at[idxb.at[pi].at[0]],
                                      rows_v.at[pr], gsems[pr]).wait()
                pltpu.async_copy(rows_v.at[pr],
                                 acc.at[idxb.at[pi].at[1]], ssems[pr],
                                 add=True)
                if deg_fn is not None:
                    deg_fn(pi)

            @pl.when(kk + 2 < nk)
            def _():
                pltpu.async_copy(sd_hbm.at[base + kk + 2],
                                 idxb.at[(pi + 2) % 4], isem)


def _sc_layer1():
    W = D_IN
    mesh = plsc.VectorSubcoreMesh(core_axis_name="c", subcore_axis_name="s")
    out_types = (jax.ShapeDtypeStruct((2, N, W), jnp.float32),
                 jax.ShapeDtypeStruct((2 * NTILES * N,), jnp.float32))
    half = NCHUNK // 2
    nkmax = half // NTILES + 1
    scratch = [
        pltpu.VMEM((4, 2, IB), jnp.int32),
        pltpu.VMEM((2, IB, W), jnp.float32),
        pltpu.VMEM((N,), jnp.float32),
        pltpu.VMEM_SHARED((N, W), jnp.float32),
        pltpu.SemaphoreType.DMA,
        pltpu.SemaphoreType.DMA,
        pltpu.SemaphoreType.DMA,
        pltpu.SemaphoreType.DMA,
        pltpu.SemaphoreType.DMA,
    ]

    def body(table, sd_hbm, zW_hbm, out, deg_out, idxb, rows_v, degp,
             acc, isem, gsem0, gsem1, ssem0, ssem1):
        c = lax.axis_index("c")
        s = lax.axis_index("s")
        r0 = s * ROWS_MAIN

        def rowcopy(src_ref, dst_ref):
            @pl.when(s < NTILES - 1)
            def _():
                pltpu.sync_copy(src_ref.at[pl.ds(r0, ROWS_MAIN)],
                                dst_ref.at[pl.ds(r0, ROWS_MAIN)])

            @pl.when(s == NTILES - 1)
            def _():
                pltpu.sync_copy(src_ref.at[pl.ds(r0, ROWS_LAST)],
                                dst_ref.at[pl.ds(r0, ROWS_LAST)])

        @pl.loop(0, N // 16)
        def _(i):
            degp[pl.ds(i * 16, 16)] = jnp.zeros((16,), jnp.float32)

        rowcopy(zW_hbm, acc)
        plsc.subcore_barrier()

        base = c * half + s * (half // NTILES) + jnp.minimum(s, 2)
        nk = jnp.where(s < 2, nkmax, nkmax - 1)

        def deg_fn(p):
            @pl.loop(0, IB // 16)
            def _(j):
                idx = idxb[p, 1, pl.ds(j * 16, 16)]
                plsc.addupdate_scatter(degp, [idx],
                                       jnp.ones((16,), jnp.float32))

        _pipelined_segsum(table, sd_hbm, idxb, rows_v, isem,
                          (gsem0, gsem1, ssem0, ssem1), acc, base, nk,
                          nkmax, deg_fn)

        plsc.subcore_barrier()
        rowcopy(acc, out.at[c])
        wid = c * NTILES + s
        pltpu.sync_copy(degp, deg_out.at[pl.ds(wid * N, N)])

    return functools.partial(pl.kernel, mesh=mesh, out_type=out_types,
                             scratch_types=scratch,
                             compiler_params=_sc_compiler_params())(body)


def _sc_layer2():
    W = H1 // 2
    mesh = plsc.VectorSubcoreMesh(core_axis_name="c", subcore_axis_name="s")
    out_types = (jax.ShapeDtypeStruct((N, W), jnp.float32),
                 jax.ShapeDtypeStruct((N, W), jnp.float32))
    nkmax = NCHUNK // NTILES + 1
    scratch = [
        pltpu.VMEM((4, 2, IB), jnp.int32),
        pltpu.VMEM((2, IB, W), jnp.float32),
        pltpu.VMEM_SHARED((N, W), jnp.float32),
        pltpu.SemaphoreType.DMA,
        pltpu.SemaphoreType.DMA,
        pltpu.SemaphoreType.DMA,
        pltpu.SemaphoreType.DMA,
        pltpu.SemaphoreType.DMA,
    ]

    def body(tA, tB, sd_hbm, zW_hbm, outA, outB, idxb, rows_v, acc,
             isem, gsem0, gsem1, ssem0, ssem1):
        c = lax.axis_index("c")
        s = lax.axis_index("s")
        r0 = s * ROWS_MAIN

        def rowcopy(src_ref, dst_ref):
            @pl.when(s < NTILES - 1)
            def _():
                pltpu.sync_copy(src_ref.at[pl.ds(r0, ROWS_MAIN)],
                                dst_ref.at[pl.ds(r0, ROWS_MAIN)])

            @pl.when(s == NTILES - 1)
            def _():
                pltpu.sync_copy(src_ref.at[pl.ds(r0, ROWS_LAST)],
                                dst_ref.at[pl.ds(r0, ROWS_LAST)])

        base = s * (NCHUNK // NTILES) + jnp.minimum(s, 4)
        nk = jnp.where(s < 4, nkmax, nkmax - 1)

        def core_prog(table, out):
            rowcopy(zW_hbm, acc)
            plsc.subcore_barrier()

            _pipelined_segsum(table, sd_hbm, idxb, rows_v, isem,
                              (gsem0, gsem1, ssem0, ssem1), acc,
                              base, nk, nkmax)

            plsc.subcore_barrier()
            rowcopy(acc, out)

        @pl.when(c == 0)
        def _():
            core_prog(tA, outA)

        @pl.when(c == 1)
        def _():
            core_prog(tB, outB)

    return functools.partial(pl.kernel, mesh=mesh, out_type=out_types,
                             scratch_types=scratch,
                             compiler_params=_sc_compiler_params())(body)


def _tc_layer1(x, part, dpart, W1l, W1r, b1):
    RB = 2000

    def body(x_r, p_r, d_r, wl_r, wr_r, b_r, hA_r, hB_r):
        deg = jnp.sum(d_r[...], axis=1, keepdims=True)
        inv = 1.0 / jnp.maximum(deg, 1.0)
        mean = (p_r[0] + p_r[1]) * inv
        h = (jnp.dot(mean, wl_r[...], preferred_element_type=jnp.float32)
             + jnp.dot(x_r[...], wr_r[...], preferred_element_type=jnp.float32)
             + b_r[...])
        h = jnp.maximum(h, 0.0)
        hA_r[...] = h[:, :H1 // 2]
        hB_r[...] = h[:, H1 // 2:]

    grid = (N // RB,)
    row_spec = lambda w: pl.BlockSpec((RB, w), lambda i: (i, 0))
    stk_spec = lambda w: pl.BlockSpec((2, RB, w), lambda i: (0, i, 0))
    full_spec = lambda a, b: pl.BlockSpec((a, b), lambda i: (0, 0))
    return pl.pallas_call(
        body,
        grid=grid,
        in_specs=[row_spec(D_IN), stk_spec(D_IN), row_spec(2 * NTILES),
                  full_spec(D_IN, H1), full_spec(D_IN, H1), full_spec(1, H1)],
        out_specs=[row_spec(H1 // 2), row_spec(H1 // 2)],
        out_shape=[jax.ShapeDtypeStruct((N, H1 // 2), jnp.float32),
                   jax.ShapeDtypeStruct((N, H1 // 2), jnp.float32)],
    )(x, part, dpart, W1l, W1r, b1)


def _tc_layer2_final(hA, hB, aA, aB, dpart, W2l, W2r, b2, Wlin, blin):
    RB = 2000

    def body(hA_r, hB_r, aA_r, aB_r, d_r, wl_r, wr_r, b_r,
             wlin_r, blin_r, out_r):
        deg = jnp.sum(d_r[...], axis=1, keepdims=True)
        inv = 1.0 / jnp.maximum(deg, 1.0)
        mean = jnp.concatenate([aA_r[...], aB_r[...]], axis=1) * inv
        h1 = jnp.concatenate([hA_r[...], hB_r[...]], axis=1)
        h2 = (jnp.dot(mean, wl_r[...], preferred_element_type=jnp.float32)
              + jnp.dot(h1, wr_r[...], preferred_element_type=jnp.float32)
              + b_r[...])
        h2 = jnp.maximum(h2, 0.0)
        o = jnp.dot(h2, wlin_r[...], preferred_element_type=jnp.float32) \
            + blin_r[...]
        m = jnp.max(o, axis=1, keepdims=True)
        lse = m + jnp.log(jnp.sum(jnp.exp(o - m), axis=1, keepdims=True))
        out_r[...] = o - lse

    grid = (N // RB,)
    row_spec = lambda w: pl.BlockSpec((RB, w), lambda i: (i, 0))
    stk_spec = lambda w: pl.BlockSpec((2, RB, w), lambda i: (0, i, 0))
    full_spec = lambda a, b: pl.BlockSpec((a, b), lambda i: (0, 0))
    return pl.pallas_call(
        body,
        grid=grid,
        in_specs=[row_spec(H1 // 2), row_spec(H1 // 2),
                  row_spec(H2 // 2), row_spec(H2 // 2), row_spec(2 * NTILES),
                  full_spec(H1, H2), full_spec(H1, H2), full_spec(1, H2),
                  full_spec(H2, OUT), full_spec(1, OUT)],
        out_specs=row_spec(OUT),
        out_shape=jax.ShapeDtypeStruct((N, OUT), jnp.float32),
    )(hA, hB, aA, aB, dpart, W2l, W2r, b2, Wlin, blin)


def kernel(x, edge_index, W1l, W1r, b1, W2l, W2r, b2, Wlin, blin):
    src = edge_index[0].astype(jnp.int32)
    dst = edge_index[1].astype(jnp.int32)
    sd = jnp.stack([src.reshape(NCHUNK, IB), dst.reshape(NCHUNK, IB)], axis=1)
    sd = jnp.concatenate(
        [sd, jnp.zeros((NCHUNK_PAD - NCHUNK, 2, IB), jnp.int32)], axis=0)

    z128 = jnp.zeros((N, 128), jnp.float32)
    z16 = jnp.zeros((N, 16), jnp.float32)

    part, dego = _sc_layer1()(x, sd, z128)
    dpart = dego.reshape(2 * NTILES, N).T

    hA, hB = _tc_layer1(x, part, dpart, W1l, W1r, b1.reshape(1, H1))

    agg2A, agg2B = _sc_layer2()(hA, hB, sd, z128)

    return _tc_layer2_final(hA, hB, agg2A, agg2B, dpart, W2l, W2r,
                            b2.reshape(1, H2), Wlin, blin.reshape(1, OUT))
